# baseline (speedup 1.0000x reference)
import functools

import jax
import jax.numpy as jnp
from jax import lax
from jax.experimental import pallas as pl
from jax.experimental.pallas import tpu as pltpu
from jax.experimental.pallas import tpu_sc as plsc

D_MODEL = 64
SCALE = float(D_MODEL) ** 0.5

_INFO = plsc.get_sparse_core_info()
_NC = _INFO.num_cores
_NS = _INFO.num_subcores
_NW = _NC * _NS
_LANES = _INFO.num_lanes

_IW = 128
_GRP = 2
_CHUNK = _GRP * _IW
_STAGE = 1024


@functools.partial(jax.jit, static_argnames=("n_rows",))
def _embed(x1, table, n_rows):
    rows_per_w = n_rows // _NW
    chunks = rows_per_w // _STAGE

    mesh = plsc.VectorSubcoreMesh(core_axis_name="c", subcore_axis_name="s")

    @functools.partial(
        pl.kernel,
        mesh=mesh,
        out_type=jax.ShapeDtypeStruct((n_rows, 2 * D_MODEL), jnp.float32),
        scratch_types=[
            pltpu.VMEM((_STAGE,), jnp.int32),
            pltpu.VMEM((_CHUNK, D_MODEL), jnp.float32),
            pltpu.VMEM((_CHUNK, 2 * D_MODEL), jnp.float32),
            pltpu.SemaphoreType.DMA,
        ],
        compiler_params=pltpu.CompilerParams(use_tc_tiling_on_sc=False),
    )
    def k(x_hbm, table_hbm, out_hbm, idx_v, rows_v, pack_v, gsem):
        wid = lax.axis_index("s") * _NC + lax.axis_index("c")
        base = wid * rows_per_w

        def chunk_body(t, _):
            pltpu.sync_copy(x_hbm.at[pl.ds(base + t * _STAGE, _STAGE)], idx_v)
            for s in range(_STAGE // _CHUNK):
                descs = []
                for j in range(_GRP):
                    descs.append(
                        pltpu.async_copy(
                            table_hbm.at[
                                idx_v.at[pl.ds((s * _GRP + j) * _IW, _IW)]
                            ],
                            rows_v.at[pl.ds(j * _IW, _IW)],
                            gsem,
                        )
                    )
                for d in descs:
                    d.wait()

                def pack_body(r, _):
                    for c in range(D_MODEL // _LANES):
                        sl = pl.ds(c * _LANES, _LANES)
                        pack_v[r, sl] = rows_v[r, sl] * SCALE
                    return ()

                lax.fori_loop(0, _CHUNK, pack_body, ())

                pltpu.sync_copy(
                    pack_v,
                    out_hbm.at[pl.ds(base + (t * (_STAGE // _CHUNK) + s)
                                     * _CHUNK, _CHUNK)],
                )
            return ()

        lax.fori_loop(0, chunks, chunk_body, ())

    return k(x1, table)


def kernel(x, table):
    b0, b1 = x.shape
    n_rows = b0 * b1
    x1 = x.reshape(n_rows).astype(jnp.int32)
    out = _embed(x1, table, n_rows)
    return (out.reshape(n_rows // 8, 8, 2 * D_MODEL)[:, :, :D_MODEL]
            .reshape(n_rows, D_MODEL).reshape(b0, b1, D_MODEL))


# speedup vs baseline: 1.2261x; 1.2261x over previous
import functools

import jax
import jax.numpy as jnp
from jax import lax
from jax.experimental import pallas as pl
from jax.experimental.pallas import tpu as pltpu
from jax.experimental.pallas import tpu_sc as plsc

D_MODEL = 64
SCALE = float(D_MODEL) ** 0.5

_INFO = plsc.get_sparse_core_info()
_NC = _INFO.num_cores
_NS = _INFO.num_subcores
_NW = _NC * _NS
_LANES = _INFO.num_lanes

_IW = 128
_GRP = 2
_CHUNK = _GRP * _IW
_STAGE = 1024


@functools.partial(jax.jit, static_argnames=("n_rows",))
def _embed(x1, table, n_rows):
    rows_per_w = n_rows // _NW
    chunks = rows_per_w // _STAGE

    mesh = plsc.VectorSubcoreMesh(core_axis_name="c", subcore_axis_name="s")

    @functools.partial(
        pl.kernel,
        mesh=mesh,
        out_type=jax.ShapeDtypeStruct((n_rows // 2, 2 * D_MODEL), jnp.float32),
        scratch_types=[
            pltpu.VMEM((_STAGE,), jnp.int32),
            pltpu.VMEM((_CHUNK, D_MODEL), jnp.float32),
            pltpu.VMEM((_CHUNK, D_MODEL), jnp.float32),
            pltpu.VMEM((_CHUNK // 2, 2 * D_MODEL), jnp.float32),
            pltpu.VMEM((_CHUNK // 2, 2 * D_MODEL), jnp.float32),
            pltpu.SemaphoreType.DMA,
            pltpu.SemaphoreType.DMA,
            pltpu.SemaphoreType.DMA,
            pltpu.SemaphoreType.DMA,
        ],
        compiler_params=pltpu.CompilerParams(use_tc_tiling_on_sc=False),
    )
    def k(x_hbm, table_hbm, out_hbm, idx_v, rows_a, rows_b, pack_a, pack_b,
          gsem_a, gsem_b, wsem_a, wsem_b):
        wid = lax.axis_index("s") * _NC + lax.axis_index("c")
        base = wid * rows_per_w
        pair_base = wid * (rows_per_w // 2)
        rows_bufs = (rows_a, rows_b)
        pack_bufs = (pack_a, pack_b)
        gsems = (gsem_a, gsem_b)
        wsems = (wsem_a, wsem_b)
        subs = _STAGE // _CHUNK
        n_sub = chunks * subs

        def fire(i, par):
            t, s = i // subs, i % subs
            rv, gs = rows_bufs[par], gsems[par]
            descs = []
            for j in range(_GRP):
                descs.append(
                    pltpu.async_copy(
                        table_hbm.at[
                            idx_v.at[pl.ds((s * _GRP + j) * _IW, _IW)]
                        ],
                        rv.at[pl.ds(j * _IW, _IW)],
                        gs,
                    )
                )
            return descs

        def stage_idx(t):
            pltpu.sync_copy(x_hbm.at[pl.ds(base + t * _STAGE, _STAGE)], idx_v)

        stage_idx(0)
        d0 = fire(0, 0)

        def handle(i, par):
            nxt = 1 - par
            rv, pv = rows_bufs[par], pack_bufs[par]

            pltpu.make_async_copy(
                table_hbm.at[idx_v.at[pl.ds(0, _IW)]],
                rv.at[pl.ds(0, _IW)], gsems[par],
            ).wait()
            pltpu.make_async_copy(
                table_hbm.at[idx_v.at[pl.ds(0, _IW)]],
                rv.at[pl.ds(_IW, _IW)], gsems[par],
            ).wait()

            @pl.when(jnp.logical_and((i + 1) % subs == 0, i + 1 < n_sub))
            def _():
                stage_idx((i + 1) // subs)

            @pl.when(i + 1 < n_sub)
            def _():
                for j in range(_GRP):
                    pltpu.async_copy(
                        table_hbm.at[
                            idx_v.at[
                                pl.ds((((i + 1) % subs) * _GRP + j) * _IW,
                                      _IW)
                            ]
                        ],
                        rows_bufs[nxt].at[pl.ds(j * _IW, _IW)],
                        gsems[nxt],
                    )

            @pl.when(i >= 2)
            def _():
                pltpu.make_async_copy(
                    pv, out_hbm.at[pl.ds(pair_base, _CHUNK // 2)], wsems[par]
                ).wait()

            def pack_body(p, _):
                for rr in range(2):
                    for c in range(D_MODEL // _LANES):
                        src = pl.ds(c * _LANES, _LANES)
                        dst = pl.ds(rr * D_MODEL + c * _LANES, _LANES)
                        pv[p, dst] = rv[2 * p + rr, src] * SCALE
                return ()

            lax.fori_loop(0, _CHUNK // 2, pack_body, ())

            pltpu.async_copy(
                pv,
                out_hbm.at[pl.ds(pair_base + i * (_CHUNK // 2), _CHUNK // 2)],
                wsems[par],
            )

        def pair_body(h, _):
            handle(2 * h, 0)
            handle(2 * h + 1, 1)
            return ()

        lax.fori_loop(0, n_sub // 2, pair_body, ())
        del d0

        for par in range(2):
            pltpu.make_async_copy(
                pack_bufs[par],
                out_hbm.at[pl.ds(pair_base, _CHUNK // 2)],
                wsems[par],
            ).wait()

    return k(x1, table)


def kernel(x, table):
    b0, b1 = x.shape
    n_rows = b0 * b1
    x1 = x.reshape(n_rows).astype(jnp.int32)
    out = _embed(x1, table, n_rows)
    return out.reshape(n_rows, D_MODEL).reshape(b0, b1, D_MODEL)
